# baseline (speedup 1.0000x reference)
import functools
import jax
import jax.numpy as jnp
import numpy as np
from jax.experimental import pallas as pl

EPSV = 1e-6
KNB = 20
HEADC = 16
NSLOPE = 0.2
NEG_BIG = -1e30


def _fused_block(x_ref, y_ref, w1f_ref, w1d_ref, w2f_ref, w2d_ref, out_ref,
                 *, nb: int, n_total: int, n_ch: int):
    j_blk = pl.program_id(1)
    yc = y_ref[0]
    xb = x_ref[0]
    yb = y_ref[0, :, pl.ds(j_blk * nb, nb)]

    inner = 2.0 * jax.lax.dot_general(
        yb, yc, (((0,), (0,)), ((), ())),
        preferred_element_type=jnp.float32,
        precision=jax.lax.Precision.HIGHEST)
    yy_all = jnp.sum(yc * yc, axis=0, keepdims=True)
    yy_blk = jnp.sum(yb * yb, axis=0)[:, None]
    pw = inner - yy_all - yy_blk

    yc1 = jnp.concatenate([yc, jnp.ones((1, n_total), jnp.float32)], axis=0)
    dds, dys, dxs, dcs = [], [], [], []
    for _ in range(KNB):
        m = jnp.max(pw, axis=1, keepdims=True)
        eqf = jnp.where(pw == m, 1.0, 0.0)
        pw = pw - eqf * (-NEG_BIG)
        nbr4 = jax.lax.dot_general(
            yc1, eqf, (((1,), (1,)), ((), ())),
            preferred_element_type=jnp.float32,
            precision=jax.lax.Precision.HIGHEST)
        nbr = nbr4[:3] / nbr4[3][None, :]
        D = nbr - yb
        dds.append(jnp.sum(D * D, axis=0))
        dys.append(jnp.sum(D * yb, axis=0))
        dxs.append(jnp.sum(D * xb, axis=0))
        dcs.append(D)

    dd = jnp.stack(dds)[:, None, :]
    dy = jnp.stack(dys)[:, None, :]
    dx = jnp.stack(dxs)[:, None, :]
    yyv = jnp.sum(yb * yb, axis=0)[None, None, :]
    yxv = jnp.sum(yb * xb, axis=0)[None, None, :]

    a1 = w1f_ref[:, 0][:, None]
    e1 = w1d_ref[:, 0][:, None]
    xxv = jnp.sum(xb * xb, axis=0)[None, :]
    dot1 = (a1 * e1) * xxv
    dsq1 = (e1 * e1) * xxv
    gam1 = (1.0 - NSLOPE) * jnp.minimum(dot1, 0.0) / (dsq1 + EPSV)
    t = a1 - gam1 * e1
    denom = jnp.maximum(jnp.sqrt(xxv) * jnp.sqrt(jnp.sum(t * t, axis=0,
                                                         keepdims=True)), 1e-12)
    g = (t / denom)[None]

    a2 = w2f_ref[:, 0][None, :, None]
    b2 = w2f_ref[:, 1][None, :, None]
    e2 = w2d_ref[:, 0][None, :, None]
    f2 = w2d_ref[:, 1][None, :, None]

    dot2 = (a2 * e2) * dd + (a2 * f2 + b2 * e2) * dy + (b2 * f2) * yyv
    dsq2 = (e2 * e2) * dd + (2.0 * e2 * f2) * dy + (f2 * f2) * yyv
    gam2 = (1.0 - NSLOPE) * jnp.minimum(dot2, 0.0) / (dsq2 + EPSV)
    al = a2 - gam2 * e2
    be = b2 - gam2 * f2
    nsq = (al * al) * dd + (2.0 * al * be) * dy + (be * be) * yyv
    ssq = jnp.sum(nsq, axis=1, keepdims=True)
    r_s = 1.0 / jnp.maximum(jnp.sqrt(ssq), 1e-12)

    n_h = n_ch // HEADC
    scale = 1.0 / np.sqrt(3.0 * HEADC)
    ga = g * al
    gb = g * be
    ga_h = jnp.concatenate(
        [jnp.sum(ga[:, h * HEADC:(h + 1) * HEADC, :], axis=1, keepdims=True)
         for h in range(n_h)], axis=1)
    gb_h = jnp.concatenate(
        [jnp.sum(gb[:, h * HEADC:(h + 1) * HEADC, :], axis=1, keepdims=True)
         for h in range(n_h)], axis=1)
    att = (ga_h * dx + gb_h * yxv) * (r_s * scale)
    w = jnp.exp(att)
    w = w / jnp.sum(w, axis=0, keepdims=True)
    w_c = jnp.concatenate(
        [jnp.broadcast_to(w[:, h:h + 1, :], (KNB, HEADC, nb))
         for h in range(n_h)], axis=1)

    wa = w_c * al
    wb_sum = jnp.sum(w_c * be, axis=0)
    for comp in range(3):
        dcomp = jnp.stack([dcs[j][comp] for j in range(KNB)])[:, None, :]
        o = jnp.sum(wa * dcomp, axis=0) + wb_sum * yb[comp][None, :]
        out_ref[0, :, comp, :] = o


@jax.jit
def kernel(x, y, W1_feat, W1_dir, W2_feat, W2_dir):
    B, _, N = x.shape
    C = W1_feat.shape[0]
    NB = 128
    grid = (B, N // NB)
    f = functools.partial(_fused_block, nb=NB, n_total=N, n_ch=C)
    return pl.pallas_call(
        f,
        grid=grid,
        in_specs=[
            pl.BlockSpec((1, 3, NB), lambda b, j: (b, 0, j)),
            pl.BlockSpec((1, 3, N), lambda b, j: (b, 0, 0)),
            pl.BlockSpec((C, 1), lambda b, j: (0, 0)),
            pl.BlockSpec((C, 1), lambda b, j: (0, 0)),
            pl.BlockSpec((C, 2), lambda b, j: (0, 0)),
            pl.BlockSpec((C, 2), lambda b, j: (0, 0)),
        ],
        out_specs=pl.BlockSpec((1, C, 3, NB), lambda b, j: (b, 0, 0, j)),
        out_shape=jax.ShapeDtypeStruct((B, C, 3, N), jnp.float32),
    )(x, y, W1_feat, W1_dir, W2_feat, W2_dir)


# speedup vs baseline: 2.3734x; 2.3734x over previous
import functools
import jax
import jax.numpy as jnp
import numpy as np
from jax.experimental import pallas as pl

EPSV = 1e-6
KNB = 20
HEADC = 16
NSLOPE = 0.2
NEG_BIG = -1e30


def _fused_block(x_ref, y_ref, w1f_ref, w1d_ref, w2f_ref, w2d_ref, out_ref,
                 *, nb: int, n_total: int, n_ch: int):
    j_blk = pl.program_id(1)
    yc = y_ref[0]
    xb = x_ref[0]
    yb = y_ref[0, :, pl.ds(j_blk * nb, nb)]

    inner = 2.0 * jax.lax.dot_general(
        yb, yc, (((0,), (0,)), ((), ())),
        preferred_element_type=jnp.float32)
    yy_all = jnp.sum(yc * yc, axis=0, keepdims=True)
    yy_blk = jnp.sum(yb * yb, axis=0)[:, None]
    pw = inner - yy_all - yy_blk

    yc1 = jnp.concatenate([yc, jnp.ones((1, n_total), jnp.float32)], axis=0)
    yc1_hi = yc1.astype(jnp.bfloat16).astype(jnp.float32)
    yc1_lo = yc1 - yc1_hi
    dds, dys, dxs, dcs = [], [], [], []
    for _ in range(KNB):
        m = jnp.max(pw, axis=1, keepdims=True)
        eqf = jnp.where(pw == m, 1.0, 0.0)
        pw = pw - eqf * (-NEG_BIG)
        nbr4 = jax.lax.dot_general(
            yc1_hi, eqf, (((1,), (1,)), ((), ())),
            preferred_element_type=jnp.float32) + jax.lax.dot_general(
            yc1_lo, eqf, (((1,), (1,)), ((), ())),
            preferred_element_type=jnp.float32)
        nbr = nbr4[:3] / nbr4[3][None, :]
        D = nbr - yb
        dds.append(jnp.sum(D * D, axis=0))
        dys.append(jnp.sum(D * yb, axis=0))
        dxs.append(jnp.sum(D * xb, axis=0))
        dcs.append(D)

    dd = jnp.stack(dds)[:, None, :]
    dy = jnp.stack(dys)[:, None, :]
    dx = jnp.stack(dxs)[:, None, :]
    yyv = jnp.sum(yb * yb, axis=0)[None, None, :]
    yxv = jnp.sum(yb * xb, axis=0)[None, None, :]

    a1 = w1f_ref[:, 0][:, None]
    e1 = w1d_ref[:, 0][:, None]
    xxv = jnp.sum(xb * xb, axis=0)[None, :]
    dot1 = (a1 * e1) * xxv
    dsq1 = (e1 * e1) * xxv
    gam1 = (1.0 - NSLOPE) * jnp.minimum(dot1, 0.0) / (dsq1 + EPSV)
    t = a1 - gam1 * e1
    denom = jnp.maximum(jnp.sqrt(xxv) * jnp.sqrt(jnp.sum(t * t, axis=0,
                                                         keepdims=True)), 1e-12)
    g = (t / denom)[None]

    a2 = w2f_ref[:, 0][None, :, None]
    b2 = w2f_ref[:, 1][None, :, None]
    e2 = w2d_ref[:, 0][None, :, None]
    f2 = w2d_ref[:, 1][None, :, None]

    dot2 = (a2 * e2) * dd + (a2 * f2 + b2 * e2) * dy + (b2 * f2) * yyv
    dsq2 = (e2 * e2) * dd + (2.0 * e2 * f2) * dy + (f2 * f2) * yyv
    gam2 = (1.0 - NSLOPE) * jnp.minimum(dot2, 0.0) / (dsq2 + EPSV)
    al = a2 - gam2 * e2
    be = b2 - gam2 * f2
    nsq = (al * al) * dd + (2.0 * al * be) * dy + (be * be) * yyv
    ssq = jnp.sum(nsq, axis=1, keepdims=True)
    r_s = 1.0 / jnp.maximum(jnp.sqrt(ssq), 1e-12)

    n_h = n_ch // HEADC
    scale = 1.0 / np.sqrt(3.0 * HEADC)
    ga = g * al
    gb = g * be
    ga_h = jnp.concatenate(
        [jnp.sum(ga[:, h * HEADC:(h + 1) * HEADC, :], axis=1, keepdims=True)
         for h in range(n_h)], axis=1)
    gb_h = jnp.concatenate(
        [jnp.sum(gb[:, h * HEADC:(h + 1) * HEADC, :], axis=1, keepdims=True)
         for h in range(n_h)], axis=1)
    att = (ga_h * dx + gb_h * yxv) * (r_s * scale)
    w = jnp.exp(att)
    w = w / jnp.sum(w, axis=0, keepdims=True)
    w_c = jnp.concatenate(
        [jnp.broadcast_to(w[:, h:h + 1, :], (KNB, HEADC, nb))
         for h in range(n_h)], axis=1)

    wa = w_c * al
    wb_sum = jnp.sum(w_c * be, axis=0)
    for comp in range(3):
        dcomp = jnp.stack([dcs[j][comp] for j in range(KNB)])[:, None, :]
        o = jnp.sum(wa * dcomp, axis=0) + wb_sum * yb[comp][None, :]
        out_ref[0, :, comp, :] = o


@jax.jit
def kernel(x, y, W1_feat, W1_dir, W2_feat, W2_dir):
    B, _, N = x.shape
    C = W1_feat.shape[0]
    NB = 128
    grid = (B, N // NB)
    f = functools.partial(_fused_block, nb=NB, n_total=N, n_ch=C)
    return pl.pallas_call(
        f,
        grid=grid,
        in_specs=[
            pl.BlockSpec((1, 3, NB), lambda b, j: (b, 0, j)),
            pl.BlockSpec((1, 3, N), lambda b, j: (b, 0, 0)),
            pl.BlockSpec((C, 1), lambda b, j: (0, 0)),
            pl.BlockSpec((C, 1), lambda b, j: (0, 0)),
            pl.BlockSpec((C, 2), lambda b, j: (0, 0)),
            pl.BlockSpec((C, 2), lambda b, j: (0, 0)),
        ],
        out_specs=pl.BlockSpec((1, C, 3, NB), lambda b, j: (b, 0, 0, j)),
        out_shape=jax.ShapeDtypeStruct((B, C, 3, N), jnp.float32),
    )(x, y, W1_feat, W1_dir, W2_feat, W2_dir)


# speedup vs baseline: 3.0857x; 1.3001x over previous
import functools
import jax
import jax.numpy as jnp
import numpy as np
from jax.experimental import pallas as pl

EPSV = 1e-6
KNB = 20
HEADC = 16
NSLOPE = 0.2
NEG_BIG = -1e30


def _fused_block(x_ref, y_ref, w1f_ref, w1d_ref, w2f_ref, w2d_ref, out_ref,
                 *, nb: int, n_total: int, n_ch: int):
    j_blk = pl.program_id(1)
    yc = y_ref[0]
    xb = x_ref[0]
    yb = y_ref[0, :, pl.ds(j_blk * nb, nb)]

    inner = 2.0 * jax.lax.dot_general(
        yb, yc, (((0,), (0,)), ((), ())),
        preferred_element_type=jnp.float32)
    yy_all = jnp.sum(yc * yc, axis=0, keepdims=True)
    yy_blk = jnp.sum(yb * yb, axis=0)[:, None]
    pw = inner - yy_all - yy_blk

    yc1 = jnp.concatenate([yc, jnp.ones((1, n_total), jnp.float32)], axis=0)
    eqfs = []
    eqf_prev = None
    nbr4s = []
    for _ in range(KNB):
        m = jnp.max(pw, axis=1, keepdims=True)
        eqf = jnp.where(pw == m, 1.0, 0.0)
        pw = pw - eqf * (-NEG_BIG)
        if eqf_prev is not None:
            nbr4s.append(jax.lax.dot_general(
                yc1, eqf_prev, (((1,), (1,)), ((), ())),
                preferred_element_type=jnp.float32))
        eqf_prev = eqf
    nbr4s.append(jax.lax.dot_general(
        yc1, eqf_prev, (((1,), (1,)), ((), ())),
        preferred_element_type=jnp.float32))

    dds, dys, dxs, dcs = [], [], [], []
    for nbr4 in nbr4s:
        nbr = nbr4[:3] / nbr4[3][None, :]
        D = nbr - yb
        dds.append(jnp.sum(D * D, axis=0))
        dys.append(jnp.sum(D * yb, axis=0))
        dxs.append(jnp.sum(D * xb, axis=0))
        dcs.append(D)

    dd = jnp.stack(dds)[:, None, :]
    dy = jnp.stack(dys)[:, None, :]
    dx = jnp.stack(dxs)[:, None, :]
    yyv = jnp.sum(yb * yb, axis=0)[None, None, :]
    yxv = jnp.sum(yb * xb, axis=0)[None, None, :]

    a1 = w1f_ref[:, 0][:, None]
    e1 = w1d_ref[:, 0][:, None]
    xxv = jnp.sum(xb * xb, axis=0)[None, :]
    dot1 = (a1 * e1) * xxv
    dsq1 = (e1 * e1) * xxv
    gam1 = (1.0 - NSLOPE) * jnp.minimum(dot1, 0.0) / (dsq1 + EPSV)
    t = a1 - gam1 * e1
    denom = jnp.maximum(jnp.sqrt(xxv) * jnp.sqrt(jnp.sum(t * t, axis=0,
                                                         keepdims=True)), 1e-12)
    g = (t / denom)[None]

    a2 = w2f_ref[:, 0][None, :, None]
    b2 = w2f_ref[:, 1][None, :, None]
    e2 = w2d_ref[:, 0][None, :, None]
    f2 = w2d_ref[:, 1][None, :, None]

    dot2 = (a2 * e2) * dd + (a2 * f2 + b2 * e2) * dy + (b2 * f2) * yyv
    dsq2 = (e2 * e2) * dd + (2.0 * e2 * f2) * dy + (f2 * f2) * yyv
    gam2 = (1.0 - NSLOPE) * jnp.minimum(dot2, 0.0) / (dsq2 + EPSV)
    al = a2 - gam2 * e2
    be = b2 - gam2 * f2
    nsq = (al * al) * dd + (2.0 * al * be) * dy + (be * be) * yyv
    ssq = jnp.sum(nsq, axis=1, keepdims=True)
    r_s = 1.0 / jnp.maximum(jnp.sqrt(ssq), 1e-12)

    n_h = n_ch // HEADC
    scale = 1.0 / np.sqrt(3.0 * HEADC)
    ga = g * al
    gb = g * be
    ga_h = jnp.concatenate(
        [jnp.sum(ga[:, h * HEADC:(h + 1) * HEADC, :], axis=1, keepdims=True)
         for h in range(n_h)], axis=1)
    gb_h = jnp.concatenate(
        [jnp.sum(gb[:, h * HEADC:(h + 1) * HEADC, :], axis=1, keepdims=True)
         for h in range(n_h)], axis=1)
    att = (ga_h * dx + gb_h * yxv) * (r_s * scale)
    w = jnp.exp(att)
    w = w / jnp.sum(w, axis=0, keepdims=True)
    w_c = jnp.concatenate(
        [jnp.broadcast_to(w[:, h:h + 1, :], (KNB, HEADC, nb))
         for h in range(n_h)], axis=1)

    wa = w_c * al
    wb_sum = jnp.sum(w_c * be, axis=0)
    for comp in range(3):
        dcomp = jnp.stack([dcs[j][comp] for j in range(KNB)])[:, None, :]
        o = jnp.sum(wa * dcomp, axis=0) + wb_sum * yb[comp][None, :]
        out_ref[0, :, comp, :] = o


@jax.jit
def kernel(x, y, W1_feat, W1_dir, W2_feat, W2_dir):
    B, _, N = x.shape
    C = W1_feat.shape[0]
    NB = 128
    grid = (B, N // NB)
    f = functools.partial(_fused_block, nb=NB, n_total=N, n_ch=C)
    return pl.pallas_call(
        f,
        grid=grid,
        in_specs=[
            pl.BlockSpec((1, 3, NB), lambda b, j: (b, 0, j)),
            pl.BlockSpec((1, 3, N), lambda b, j: (b, 0, 0)),
            pl.BlockSpec((C, 1), lambda b, j: (0, 0)),
            pl.BlockSpec((C, 1), lambda b, j: (0, 0)),
            pl.BlockSpec((C, 2), lambda b, j: (0, 0)),
            pl.BlockSpec((C, 2), lambda b, j: (0, 0)),
        ],
        out_specs=pl.BlockSpec((1, C, 3, NB), lambda b, j: (b, 0, 0, j)),
        out_shape=jax.ShapeDtypeStruct((B, C, 3, N), jnp.float32),
    )(x, y, W1_feat, W1_dir, W2_feat, W2_dir)


# speedup vs baseline: 3.0921x; 1.0021x over previous
import functools
import jax
import jax.numpy as jnp
import numpy as np
from jax.experimental import pallas as pl

EPSV = 1e-6
KNB = 20
HEADC = 16
NSLOPE = 0.2
NEG_BIG = -1e30


def _fused_block(x_ref, y_ref, w1f_ref, w1d_ref, w2f_ref, w2d_ref, out_ref,
                 *, nb: int, n_total: int, n_ch: int):
    j_blk = pl.program_id(1)
    yc = y_ref[0]
    xb = x_ref[0]
    yb = y_ref[0, :, pl.ds(j_blk * nb, nb)]

    inner = 2.0 * jax.lax.dot_general(
        yb, yc, (((0,), (0,)), ((), ())),
        preferred_element_type=jnp.float32)
    yy_all = jnp.sum(yc * yc, axis=0, keepdims=True)
    yy_blk = jnp.sum(yb * yb, axis=0)[:, None]
    pw = inner - yy_all - yy_blk

    yc1 = jnp.concatenate([yc, jnp.ones((1, n_total), jnp.float32)], axis=0)
    eqf_prev = None
    nbr4s = []
    for _ in range(KNB):
        m = jnp.max(pw, axis=1, keepdims=True)
        eqf = jnp.where(pw == m, 1.0, 0.0)
        pw = pw - eqf * (-NEG_BIG)
        if eqf_prev is not None:
            nbr4s.append(jax.lax.dot_general(
                yc1, eqf_prev, (((1,), (1,)), ((), ())),
                preferred_element_type=jnp.float32))
        eqf_prev = eqf
    nbr4s.append(jax.lax.dot_general(
        yc1, eqf_prev, (((1,), (1,)), ((), ())),
        preferred_element_type=jnp.float32))

    dds, dys, dxs, dcs = [], [], [], []
    for nbr4 in nbr4s:
        nbr = nbr4[:3] / nbr4[3][None, :]
        D = nbr - yb
        dds.append(jnp.sum(D * D, axis=0))
        dys.append(jnp.sum(D * yb, axis=0))
        dxs.append(jnp.sum(D * xb, axis=0))
        dcs.append(D)

    dd = jnp.stack(dds)[:, None, :]
    dy = jnp.stack(dys)[:, None, :]
    dx = jnp.stack(dxs)[:, None, :]
    yyv = jnp.sum(yb * yb, axis=0)[None, None, :]
    yxv = jnp.sum(yb * xb, axis=0)[None, None, :]

    a1 = w1f_ref[:, 0][:, None]
    e1 = w1d_ref[:, 0][:, None]
    xxv = jnp.sum(xb * xb, axis=0)[None, :]
    dot1 = (a1 * e1) * xxv
    dsq1 = (e1 * e1) * xxv
    gam1 = (1.0 - NSLOPE) * jnp.minimum(dot1, 0.0) / (dsq1 + EPSV)
    t = a1 - gam1 * e1
    denom = jnp.maximum(jnp.sqrt(xxv) * jnp.sqrt(jnp.sum(t * t, axis=0,
                                                         keepdims=True)), 1e-12)
    g = (t / denom)[None]

    a2 = w2f_ref[:, 0][None, :, None]
    b2 = w2f_ref[:, 1][None, :, None]
    e2 = w2d_ref[:, 0][None, :, None]
    f2 = w2d_ref[:, 1][None, :, None]

    dot2 = (a2 * e2) * dd + (a2 * f2 + b2 * e2) * dy + (b2 * f2) * yyv
    dsq2 = (e2 * e2) * dd + (2.0 * e2 * f2) * dy + (f2 * f2) * yyv
    gam2 = (1.0 - NSLOPE) * jnp.minimum(dot2, 0.0) / (dsq2 + EPSV)
    al = a2 - gam2 * e2
    be = b2 - gam2 * f2
    nsq = (al * al) * dd + (2.0 * al * be) * dy + (be * be) * yyv
    ssq = jnp.sum(nsq, axis=1, keepdims=True)
    r_s = 1.0 / jnp.maximum(jnp.sqrt(ssq), 1e-12)

    n_h = n_ch // HEADC
    scale = 1.0 / np.sqrt(3.0 * HEADC)
    ga = g * al
    gb = g * be
    ga_h = jnp.concatenate(
        [jnp.sum(ga[:, h * HEADC:(h + 1) * HEADC, :], axis=1, keepdims=True)
         for h in range(n_h)], axis=1)
    gb_h = jnp.concatenate(
        [jnp.sum(gb[:, h * HEADC:(h + 1) * HEADC, :], axis=1, keepdims=True)
         for h in range(n_h)], axis=1)
    att = (ga_h * dx + gb_h * yxv) * (r_s * scale)
    w = jnp.exp(att)
    w = w / jnp.sum(w, axis=0, keepdims=True)
    w_c = jnp.concatenate(
        [jnp.broadcast_to(w[:, h:h + 1, :], (KNB, HEADC, nb))
         for h in range(n_h)], axis=1)

    wa = w_c * al
    wb_sum = jnp.sum(w_c * be, axis=0)
    for comp in range(3):
        dcomp = jnp.stack([dcs[j][comp] for j in range(KNB)])[:, None, :]
        o = jnp.sum(wa * dcomp, axis=0) + wb_sum * yb[comp][None, :]
        out_ref[0, :, comp, :] = o


@jax.jit
def kernel(x, y, W1_feat, W1_dir, W2_feat, W2_dir):
    B, _, N = x.shape
    C = W1_feat.shape[0]
    NB = 128
    grid = (B, N // NB)
    f = functools.partial(_fused_block, nb=NB, n_total=N, n_ch=C)
    return pl.pallas_call(
        f,
        grid=grid,
        in_specs=[
            pl.BlockSpec((1, 3, NB), lambda b, j: (b, 0, j)),
            pl.BlockSpec((1, 3, N), lambda b, j: (b, 0, 0)),
            pl.BlockSpec((C, 1), lambda b, j: (0, 0)),
            pl.BlockSpec((C, 1), lambda b, j: (0, 0)),
            pl.BlockSpec((C, 2), lambda b, j: (0, 0)),
            pl.BlockSpec((C, 2), lambda b, j: (0, 0)),
        ],
        out_specs=pl.BlockSpec((1, C, 3, NB), lambda b, j: (b, 0, 0, j)),
        out_shape=jax.ShapeDtypeStruct((B, C, 3, N), jnp.float32),
    )(x, y, W1_feat, W1_dir, W2_feat, W2_dir)
